# baseline (speedup 1.0000x reference)
import functools

import jax
import jax.numpy as jnp
from jax import lax
from jax.experimental import pallas as pl
from jax.experimental.pallas import tpu as pltpu
from jax.experimental.pallas import tpu_sc as plsc

EMBED_DIM = 1152
NUM_TOKENS = 4 * 2048

_info = plsc.get_sparse_core_info()
_NC = _info.num_cores
_NS = _info.num_subcores
_NW = _NC * _NS
_BPW = NUM_TOKENS // _NW
_CH = 8
_NCHUNK = _BPW // _CH
_NBUF = 8
_AHEAD = _NBUF // 2

_mesh = plsc.VectorSubcoreMesh(core_axis_name="c", subcore_axis_name="s")


@functools.partial(
    pl.kernel,
    out_type=jax.ShapeDtypeStruct((NUM_TOKENS, EMBED_DIM), jnp.float32),
    mesh=_mesh,
    scratch_types=[
        pltpu.VMEM((_BPW,), jnp.int32),
        pltpu.VMEM((_NBUF, _CH, EMBED_DIM), jnp.float32),
        pltpu.SemaphoreType.DMA((_NBUF,)),
        pltpu.SemaphoreType.DMA((_NBUF,)),
    ],
)
def _gather_kernel(idx_hbm, table_hbm, out_hbm, idx_v, rows_v, gsem, ssem):
    wid = lax.axis_index("s") * _NC + lax.axis_index("c")
    base = wid * _BPW
    pltpu.sync_copy(idx_hbm.at[pl.ds(base, _BPW)], idx_v)

    def fire_gather(c):
        return pltpu.async_copy(
            table_hbm.at[idx_v.at[pl.ds(c * _CH, _CH)]],
            rows_v.at[c % _NBUF],
            gsem.at[c % _NBUF],
        )

    def fire_store(c):
        return pltpu.async_copy(
            rows_v.at[c % _NBUF],
            out_hbm.at[pl.ds(base + c * _CH, _CH)],
            ssem.at[c % _NBUF],
        )

    gathers = [None] * _NCHUNK
    stores = [None] * _NCHUNK
    for c in range(_AHEAD):
        gathers[c] = fire_gather(c)
    for c in range(_NCHUNK):
        n = c + _AHEAD
        if n < _NCHUNK:
            if n >= _NBUF:
                stores[n - _NBUF].wait()
            gathers[n] = fire_gather(n)
        gathers[c].wait()
        stores[c] = fire_store(c)
    for c in range(_NCHUNK - _NBUF, _NCHUNK):
        stores[c].wait()


@jax.jit
def kernel(token_ids, table):
    flat = token_ids.reshape(-1).astype(jnp.int32)
    out = _gather_kernel(flat, table)
    return out.reshape(token_ids.shape + (table.shape[1],))


# speedup vs baseline: 1.0137x; 1.0137x over previous
import functools

import jax
import jax.numpy as jnp
from jax import lax
from jax.experimental import pallas as pl
from jax.experimental.pallas import tpu as pltpu
from jax.experimental.pallas import tpu_sc as plsc

EMBED_DIM = 1152
NUM_TOKENS = 4 * 2048

_info = plsc.get_sparse_core_info()
_NC = _info.num_cores
_NS = _info.num_subcores
_NW = _NC * _NS
_BPW = NUM_TOKENS // _NW
_CH = 16
_NCHUNK = _BPW // _CH
_NBUF = 6
_AHEAD = 4

_mesh = plsc.VectorSubcoreMesh(core_axis_name="c", subcore_axis_name="s")


@functools.partial(
    pl.kernel,
    out_type=jax.ShapeDtypeStruct((NUM_TOKENS, EMBED_DIM), jnp.float32),
    mesh=_mesh,
    scratch_types=[
        pltpu.VMEM((_BPW,), jnp.int32),
        pltpu.VMEM((_NBUF, _CH, EMBED_DIM), jnp.float32),
        pltpu.SemaphoreType.DMA((_NBUF,)),
        pltpu.SemaphoreType.DMA((_NBUF,)),
    ],
)
def _gather_kernel(idx_hbm, table_hbm, out_hbm, idx_v, rows_v, gsem, ssem):
    wid = lax.axis_index("s") * _NC + lax.axis_index("c")
    base = wid * _BPW
    pltpu.sync_copy(idx_hbm.at[pl.ds(base, _BPW)], idx_v)

    def fire_gather(c):
        return pltpu.async_copy(
            table_hbm.at[idx_v.at[pl.ds(c * _CH, _CH)]],
            rows_v.at[c % _NBUF],
            gsem.at[c % _NBUF],
        )

    def fire_store(c):
        return pltpu.async_copy(
            rows_v.at[c % _NBUF],
            out_hbm.at[pl.ds(base + c * _CH, _CH)],
            ssem.at[c % _NBUF],
        )

    gathers = [None] * _NCHUNK
    stores = [None] * _NCHUNK
    for c in range(_AHEAD):
        gathers[c] = fire_gather(c)
    for c in range(_NCHUNK):
        n = c + _AHEAD
        if n < _NCHUNK:
            if n >= _NBUF:
                stores[n - _NBUF].wait()
            gathers[n] = fire_gather(n)
        gathers[c].wait()
        stores[c] = fire_store(c)
    for c in range(_NCHUNK - _NBUF, _NCHUNK):
        stores[c].wait()


@jax.jit
def kernel(token_ids, table):
    flat = token_ids.reshape(-1).astype(jnp.int32)
    out = _gather_kernel(flat, table)
    return out.reshape(token_ids.shape + (table.shape[1],))


# speedup vs baseline: 1.0245x; 1.0107x over previous
import functools

import jax
import jax.numpy as jnp
from jax import lax
from jax.experimental import pallas as pl
from jax.experimental.pallas import tpu as pltpu
from jax.experimental.pallas import tpu_sc as plsc

EMBED_DIM = 1152
NUM_TOKENS = 4 * 2048

_info = plsc.get_sparse_core_info()
_NC = _info.num_cores
_NS = _info.num_subcores
_NW = _NC * _NS
_BPW = NUM_TOKENS // _NW
_CH = 32
_NCHUNK = _BPW // _CH
_NBUF = 3
_AHEAD = 2

_mesh = plsc.VectorSubcoreMesh(core_axis_name="c", subcore_axis_name="s")


@functools.partial(
    pl.kernel,
    out_type=jax.ShapeDtypeStruct((NUM_TOKENS, EMBED_DIM), jnp.float32),
    mesh=_mesh,
    scratch_types=[
        pltpu.VMEM((_BPW,), jnp.int32),
        pltpu.VMEM((_NBUF, _CH, EMBED_DIM), jnp.float32),
        pltpu.SemaphoreType.DMA((_NBUF,)),
        pltpu.SemaphoreType.DMA((_NBUF,)),
    ],
)
def _gather_kernel(idx_hbm, table_hbm, out_hbm, idx_v, rows_v, gsem, ssem):
    wid = lax.axis_index("s") * _NC + lax.axis_index("c")
    base = wid * _BPW
    pltpu.sync_copy(idx_hbm.at[pl.ds(base, _BPW)], idx_v)

    def fire_gather(c):
        return pltpu.async_copy(
            table_hbm.at[idx_v.at[pl.ds(c * _CH, _CH)]],
            rows_v.at[c % _NBUF],
            gsem.at[c % _NBUF],
        )

    def fire_store(c):
        return pltpu.async_copy(
            rows_v.at[c % _NBUF],
            out_hbm.at[pl.ds(base + c * _CH, _CH)],
            ssem.at[c % _NBUF],
        )

    gathers = [None] * _NCHUNK
    stores = [None] * _NCHUNK
    for c in range(_AHEAD):
        gathers[c] = fire_gather(c)
    for c in range(_NCHUNK):
        n = c + _AHEAD
        if n < _NCHUNK:
            if n >= _NBUF:
                stores[n - _NBUF].wait()
            gathers[n] = fire_gather(n)
        gathers[c].wait()
        stores[c] = fire_store(c)
    for c in range(_NCHUNK - _NBUF, _NCHUNK):
        stores[c].wait()


@jax.jit
def kernel(token_ids, table):
    flat = token_ids.reshape(-1).astype(jnp.int32)
    out = _gather_kernel(flat, table)
    return out.reshape(token_ids.shape + (table.shape[1],))
